# Initial kernel scaffold; baseline (speedup 1.0000x reference)
import functools

import jax
import jax.numpy as jnp
from jax import lax
from jax.experimental import pallas as pl
from jax.experimental.pallas import tpu as pltpu
from jax.experimental.pallas import tpu_sc as plsc

B, L, D = 4096, 200, 64
N = B * L
NC, NS = 2, 16
NW = NC * NS
PER_W = N // NW
CHUNK = 512
SUB = 128
NSUB = CHUNK // SUB
NCHUNK = PER_W // CHUNK

_mesh = plsc.VectorSubcoreMesh(core_axis_name="c", subcore_axis_name="s")


@functools.partial(
    pl.kernel,
    mesh=_mesh,
    out_type=jax.ShapeDtypeStruct((N, D), jnp.float32),
    scratch_types=[
        pltpu.VMEM((CHUNK,), jnp.int32),
        pltpu.VMEM((CHUNK,), jnp.int32),
        pltpu.VMEM((CHUNK, D), jnp.float32),
        pltpu.VMEM((CHUNK, D), jnp.float32),
        pltpu.SemaphoreType.DMA,
    ],
)
def _embed_kernel(x_hbm, seq_hbm, tab_hbm, pos_hbm, out_hbm,
                  tok_idx, pos_idx, tok_rows, pos_rows, sem):
    wid = lax.axis_index("s") * NC + lax.axis_index("c")
    base0 = wid * PER_W

    def chunk_body(i, carry):
        base = base0 + i * CHUNK
        pltpu.sync_copy(x_hbm.at[pl.ds(base, CHUNK)], tok_idx)
        pltpu.sync_copy(seq_hbm.at[pl.ds(base, CHUNK)], pos_idx)
        handles = []
        for j in range(NSUB):
            s = pl.ds(j * SUB, SUB)
            handles.append(
                pltpu.async_copy(tab_hbm.at[tok_idx.at[s]], tok_rows.at[s], sem))
            handles.append(
                pltpu.async_copy(pos_hbm.at[pos_idx.at[s]], pos_rows.at[s], sem))
        for h in handles:
            h.wait()

        def add_row(r, c2):
            for c in range(D // 16):
                sl = pl.ds(c * 16, 16)
                tok_rows[r, sl] = tok_rows[r, sl] + pos_rows[r, sl]
            return c2

        lax.fori_loop(0, CHUNK, add_row, 0)
        pltpu.sync_copy(tok_rows, out_hbm.at[pl.ds(base, CHUNK)])
        return carry

    lax.fori_loop(0, NCHUNK, chunk_body, 0)


def kernel(x, seq_idx, embed_table, pos_table):
    x_flat = x.reshape(-1).astype(jnp.int32)
    seq_flat = seq_idx.reshape(-1).astype(jnp.int32)
    out = _embed_kernel(x_flat, seq_flat, embed_table, pos_table)
    return out.reshape(B, L, D)


# speedup vs baseline: 2.0074x; 2.0074x over previous
import functools

import jax
import jax.numpy as jnp
from jax import lax
from jax.experimental import pallas as pl
from jax.experimental.pallas import tpu as pltpu
from jax.experimental.pallas import tpu_sc as plsc

B, L, D = 4096, 200, 64
N = B * L
NC, NS = 2, 16
NW = NC * NS
PER_W = N // NW
CHUNK = 512
SUB = 128
NSUB = CHUNK // SUB
NCHUNK = PER_W // CHUNK

_mesh = plsc.VectorSubcoreMesh(core_axis_name="c", subcore_axis_name="s")


@functools.partial(
    pl.kernel,
    mesh=_mesh,
    out_type=jax.ShapeDtypeStruct((N, D), jnp.float32),
    compiler_params=pltpu.CompilerParams(use_tc_tiling_on_sc=False),
    scratch_types=[
        pltpu.VMEM((CHUNK,), jnp.int32),
        pltpu.VMEM((CHUNK,), jnp.int32),
        pltpu.VMEM((CHUNK, D), jnp.float32),
        pltpu.VMEM((CHUNK, D), jnp.float32),
        pltpu.SemaphoreType.DMA,
    ],
)
def _embed_kernel(x_hbm, seq_hbm, tab_hbm, pos_hbm, out_hbm,
                  tok_idx, pos_idx, tok_rows, pos_rows, sem):
    wid = lax.axis_index("s") * NC + lax.axis_index("c")
    base0 = wid * PER_W

    def chunk_body(i, carry):
        base = base0 + i * CHUNK
        pltpu.sync_copy(x_hbm.at[pl.ds(base, CHUNK)], tok_idx)
        pltpu.sync_copy(seq_hbm.at[pl.ds(base, CHUNK)], pos_idx)
        handles = []
        for j in range(NSUB):
            s = pl.ds(j * SUB, SUB)
            handles.append(
                pltpu.async_copy(tab_hbm.at[tok_idx.at[s]], tok_rows.at[s], sem))
            handles.append(
                pltpu.async_copy(pos_hbm.at[pos_idx.at[s]], pos_rows.at[s], sem))
        for h in handles:
            h.wait()

        def add_row(r, c2):
            for c in range(D // 16):
                sl = pl.ds(c * 16, 16)
                tok_rows[r, sl] = tok_rows[r, sl] + pos_rows[r, sl]
            return c2

        lax.fori_loop(0, CHUNK, add_row, 0)
        pltpu.sync_copy(tok_rows, out_hbm.at[pl.ds(base, CHUNK)])
        return carry

    lax.fori_loop(0, NCHUNK, chunk_body, 0)


def kernel(x, seq_idx, embed_table, pos_table):
    x_flat = x.reshape(-1).astype(jnp.int32)
    seq_flat = seq_idx.reshape(-1).astype(jnp.int32)
    out = _embed_kernel(x_flat, seq_flat, embed_table, pos_table)
    return out.reshape(B, L, D)
